# Initial kernel scaffold; baseline (speedup 1.0000x reference)
import jax
import jax.numpy as jnp
from jax.experimental import pallas as pl


def kernel(x, edge_index, edge_attr, We, be, W1, b1, W2, b2, gamma, beta):
    raise NotImplementedError("write your pallas kernel here")


# speedup vs baseline: 2.7673x; 2.7673x over previous
import functools

import jax
import jax.numpy as jnp
from jax import lax
from jax.experimental import pallas as pl
from jax.experimental.pallas import tpu as pltpu
from jax.experimental.pallas import tpu_sc as plsc

N_NODES = 10000
N_EDGES = 320000
HIDDEN = 128
EDGE_FEATS = 16
N_LAYERS = 4
BN_EPS = 1e-5

NC = 2
NS = 16
NW = NC * NS
E_PER_W = N_EDGES // NW
EBLK = 80
NBLK = E_PER_W // EBLK
N_PAD = 10240
ROWS_PER_TILE = N_PAD // NS
ZROWS = 128
NVH = HIDDEN // 16


def _make_mp_kernel():
    mesh = plsc.VectorSubcoreMesh(core_axis_name="c", subcore_axis_name="s")

    @functools.partial(
        pl.kernel,
        out_type=jax.ShapeDtypeStruct((NC * N_PAD, HIDDEN), jnp.float32),
        mesh=mesh,
        scratch_types=[
            pltpu.VMEM((EBLK,), jnp.int32),
            pltpu.VMEM((EBLK,), jnp.int32),
            pltpu.VMEM((EBLK, HIDDEN), jnp.float32),
            pltpu.VMEM((EBLK, HIDDEN), jnp.float32),
            pltpu.VMEM((ZROWS, HIDDEN), jnp.float32),
            pltpu.VMEM_SHARED((N_PAD, HIDDEN), jnp.float32),
            pltpu.SemaphoreType.DMA,
        ],
    )
    def mp(h_hbm, e_hbm, src_hbm, dst_hbm, out_hbm,
           src_v, dst_v, rows_v, e_v, zbuf, agg_sh, gsem):
        c = lax.axis_index("c")
        s = lax.axis_index("s")
        wid = c * NS + s
        tile_base = s * ROWS_PER_TILE

        zeros16 = jnp.zeros((16,), jnp.float32)

        def _zero_row(i, carry):
            for j in range(NVH):
                zbuf[i, pl.ds(j * 16, 16)] = zeros16
            return carry

        lax.fori_loop(0, ZROWS, _zero_row, 0)
        for r in range(ROWS_PER_TILE // ZROWS):
            pltpu.sync_copy(zbuf, agg_sh.at[pl.ds(tile_base + r * ZROWS, ZROWS)])
        plsc.subcore_barrier()

        def _block(i, carry):
            base = wid * E_PER_W + i * EBLK
            pltpu.sync_copy(src_hbm.at[pl.ds(base, EBLK)], src_v)
            pltpu.sync_copy(dst_hbm.at[pl.ds(base, EBLK)], dst_v)
            pltpu.async_copy(h_hbm.at[src_v], rows_v, gsem).wait()
            pltpu.sync_copy(e_hbm.at[pl.ds(base, EBLK)], e_v)

            def _edge(k, carry2):
                for j in range(NVH):
                    sl = pl.ds(j * 16, 16)
                    v = rows_v[k, sl] + e_v[k, sl]
                    rows_v[k, sl] = jnp.maximum(v, 0.0)
                return carry2

            lax.fori_loop(0, EBLK, _edge, 0)
            pltpu.sync_copy(rows_v, agg_sh.at[dst_v], add=True)
            return carry

        lax.fori_loop(0, NBLK, _block, 0)
        plsc.subcore_barrier()

        for r in range(ROWS_PER_TILE // ZROWS):
            row0 = tile_base + r * ZROWS
            pltpu.sync_copy(agg_sh.at[pl.ds(row0, ZROWS)],
                            out_hbm.at[pl.ds(c * N_PAD + row0, ZROWS)])

    return mp


_mp_kernel = _make_mp_kernel()


def _edge_tf_body(attr_ref, we_ref, be_ref, out_ref):
    out_ref[...] = (
        jnp.dot(attr_ref[...], we_ref[...], preferred_element_type=jnp.float32)
        + be_ref[...]
    )


def _edge_transform(edge_attr, We, be):
    blk = 4000
    grid = N_EDGES // blk
    return pl.pallas_call(
        _edge_tf_body,
        grid=(grid,),
        in_specs=[
            pl.BlockSpec((blk, EDGE_FEATS), lambda i: (i, 0)),
            pl.BlockSpec((EDGE_FEATS, HIDDEN), lambda i: (0, 0)),
            pl.BlockSpec((1, HIDDEN), lambda i: (0, 0)),
        ],
        out_specs=pl.BlockSpec((blk, HIDDEN), lambda i: (i, 0)),
        out_shape=jax.ShapeDtypeStruct((N_EDGES, HIDDEN), jnp.float32),
    )(edge_attr, We, be.reshape(1, HIDDEN))


def _dense_body(h_ref, p_ref, w1_ref, b1_ref, w2_ref, b2_ref,
                g_ref, bt_ref, o_ref):
    h = h_ref[...]
    z = h + p_ref[0:N_NODES, :] + p_ref[N_PAD:N_PAD + N_NODES, :]
    z = jnp.dot(z, w1_ref[...], preferred_element_type=jnp.float32) + b1_ref[...]
    z = jnp.maximum(z, 0.0)
    z = jnp.dot(z, w2_ref[...], preferred_element_type=jnp.float32) + b2_ref[...]
    mu = jnp.mean(z, axis=0, keepdims=True)
    var = jnp.mean(jnp.square(z - mu), axis=0, keepdims=True)
    zn = (z - mu) * lax.rsqrt(var + BN_EPS) * g_ref[...] + bt_ref[...]
    o_ref[...] = jnp.maximum(zn, 0.0) + h


def _dense_layer(h, p, W1l, b1l, W2l, b2l, gl, btl):
    spec = lambda shape: pl.BlockSpec(shape, lambda: tuple(0 for _ in shape))
    return pl.pallas_call(
        _dense_body,
        in_specs=[
            spec((N_NODES, HIDDEN)),
            spec((NC * N_PAD, HIDDEN)),
            spec((HIDDEN, HIDDEN)),
            spec((1, HIDDEN)),
            spec((HIDDEN, HIDDEN)),
            spec((1, HIDDEN)),
            spec((1, HIDDEN)),
            spec((1, HIDDEN)),
        ],
        out_specs=spec((N_NODES, HIDDEN)),
        out_shape=jax.ShapeDtypeStruct((N_NODES, HIDDEN), jnp.float32),
    )(h, p, W1l, b1l.reshape(1, HIDDEN), W2l, b2l.reshape(1, HIDDEN),
      gl.reshape(1, HIDDEN), btl.reshape(1, HIDDEN))


def kernel(x, edge_index, edge_attr, We, be, W1, b1, W2, b2, gamma, beta):
    src = edge_index[0].astype(jnp.int32)
    dst = edge_index[1].astype(jnp.int32)
    e = _edge_transform(edge_attr, We, be)
    h = x
    for i in range(N_LAYERS):
        p = _mp_kernel(h, e, src, dst)
        h = _dense_layer(h, p, W1[i], b1[i], W2[i], b2[i], gamma[i], beta[i])
    return h


# speedup vs baseline: 6.0938x; 2.2021x over previous
import functools

import jax
import jax.numpy as jnp
from jax import lax
from jax.experimental import pallas as pl
from jax.experimental.pallas import tpu as pltpu
from jax.experimental.pallas import tpu_sc as plsc

N_NODES = 10000
N_EDGES = 320000
HIDDEN = 128
EDGE_FEATS = 16
N_LAYERS = 4
BN_EPS = 1e-5

NC = 2
NS = 16
NW = NC * NS
E_PER_W = N_EDGES // NW
EBLK = 40
NBLK = E_PER_W // EBLK
N_PAD = 10240
ROWS_PER_TILE = N_PAD // NS
ZROWS = 128
NVH = HIDDEN // 16


def _make_mp_kernel():
    mesh = plsc.VectorSubcoreMesh(core_axis_name="c", subcore_axis_name="s")

    @functools.partial(
        pl.kernel,
        out_type=jax.ShapeDtypeStruct((NC * N_PAD, HIDDEN), jnp.float32),
        mesh=mesh,
        scratch_types=[
            pltpu.VMEM_SHARED((N_PAD, HIDDEN), jnp.float32),
            pltpu.VMEM((E_PER_W,), jnp.int32),
            pltpu.VMEM((EBLK,), jnp.int32),
            pltpu.VMEM((EBLK,), jnp.int32),
            pltpu.VMEM((EBLK, HIDDEN), jnp.float32),
            pltpu.VMEM((EBLK, HIDDEN), jnp.float32),
            pltpu.VMEM((EBLK, HIDDEN), jnp.float32),
            pltpu.VMEM((EBLK, HIDDEN), jnp.float32),
            pltpu.VMEM((EBLK, HIDDEN), jnp.float32),
            pltpu.VMEM((EBLK, HIDDEN), jnp.float32),
            pltpu.SemaphoreType.DMA,
            pltpu.SemaphoreType.DMA,
            pltpu.SemaphoreType.DMA,
            pltpu.SemaphoreType.DMA,
            pltpu.SemaphoreType.DMA,
            pltpu.SemaphoreType.DMA,
            pltpu.SemaphoreType.DMA,
            pltpu.SemaphoreType.DMA,
        ],
    )
    def mp(h_hbm, e_hbm, src_hbm, dst_hbm, out_hbm,
           agg_sh, src_v, dstA, dstB, rowsA, rowsB, eA, eB, msgA, msgB,
           gsA, gsB, esA, esB, ssA, ssB, dsA, dsB):
        c = lax.axis_index("c")
        s = lax.axis_index("s")
        wid = c * NS + s
        tile_base = s * ROWS_PER_TILE
        ebase = wid * E_PER_W
        slots = (
            (dstA, rowsA, eA, msgA, gsA, esA, ssA, dsA),
            (dstB, rowsB, eB, msgB, gsB, esB, ssB, dsB),
        )

        pltpu.sync_copy(src_hbm.at[pl.ds(ebase, E_PER_W)], src_v)

        zeros16 = jnp.zeros((16,), jnp.float32)

        def _zero_row(i, carry):
            for j in range(NVH):
                msgA[i, pl.ds(j * 16, 16)] = zeros16
            return carry

        lax.fori_loop(0, EBLK, _zero_row, 0)
        for r in range(ROWS_PER_TILE // EBLK):
            pltpu.sync_copy(msgA, agg_sh.at[pl.ds(tile_base + r * EBLK, EBLK)])
        plsc.subcore_barrier()

        def start_loads(i, sl):
            dst_i, rows_v, e_v, _, gs, es, _, ds_ = sl
            base = ebase + i * EBLK
            pltpu.async_copy(h_hbm.at[src_v.at[pl.ds(i * EBLK, EBLK)]],
                             rows_v, gs)
            pltpu.async_copy(e_hbm.at[pl.ds(base, EBLK)], e_v, es)
            pltpu.async_copy(dst_hbm.at[pl.ds(base, EBLK)], dst_i, ds_)

        def wait_loads(i, sl):
            dst_i, rows_v, e_v, _, gs, es, _, ds_ = sl
            base = ebase + i * EBLK
            pltpu.make_async_copy(h_hbm.at[src_v.at[pl.ds(i * EBLK, EBLK)]],
                                  rows_v, gs).wait()
            pltpu.make_async_copy(e_hbm.at[pl.ds(base, EBLK)], e_v, es).wait()
            pltpu.make_async_copy(dst_hbm.at[pl.ds(base, EBLK)], dst_i,
                                  ds_).wait()

        def compute(sl):
            _, rows_v, e_v, msg_v, _, _, _, _ = sl

            def _edge(k, carry2):
                for j in range(NVH):
                    q = pl.ds(j * 16, 16)
                    msg_v[k, q] = jnp.maximum(rows_v[k, q] + e_v[k, q], 0.0)
                return carry2

            lax.fori_loop(0, EBLK, _edge, 0)

        def start_scatter(sl):
            dst_i, _, _, msg_v, _, _, ss, _ = sl
            pltpu.async_copy(msg_v, agg_sh.at[dst_i], ss, add=True)

        def wait_scatter(sl):
            dst_i, _, _, msg_v, _, _, ss, _ = sl
            pltpu.make_async_copy(msg_v, agg_sh.at[dst_i], ss).wait()

        A, B = slots

        start_loads(0, A)
        start_loads(1, B)

        wait_loads(0, A); compute(A); start_scatter(A); start_loads(2, A)
        wait_loads(1, B); compute(B); start_scatter(B); start_loads(3, B)

        def _pair(g, carry):
            wait_scatter(A)
            wait_loads(2 * g, A)
            compute(A)
            start_scatter(A)
            start_loads(2 * g + 2, A)
            wait_scatter(B)
            wait_loads(2 * g + 1, B)
            compute(B)
            start_scatter(B)
            start_loads(2 * g + 3, B)
            return carry

        lax.fori_loop(1, NBLK // 2 - 1, _pair, 0)

        gl = NBLK // 2 - 1
        wait_scatter(A); wait_loads(2 * gl, A); compute(A); start_scatter(A)
        wait_scatter(B); wait_loads(2 * gl + 1, B); compute(B); start_scatter(B)
        wait_scatter(A)
        wait_scatter(B)
        plsc.subcore_barrier()

        pltpu.sync_copy(
            agg_sh.at[pl.ds(tile_base, ROWS_PER_TILE)],
            out_hbm.at[pl.ds(c * N_PAD + tile_base, ROWS_PER_TILE)])

    return mp


_mp_kernel = _make_mp_kernel()


def _edge_tf_body(attr_ref, we_ref, be_ref, out_ref):
    out_ref[...] = (
        jnp.dot(attr_ref[...], we_ref[...], preferred_element_type=jnp.float32)
        + be_ref[...]
    )


def _edge_transform(edge_attr, We, be):
    blk = 4000
    grid = N_EDGES // blk
    return pl.pallas_call(
        _edge_tf_body,
        grid=(grid,),
        in_specs=[
            pl.BlockSpec((blk, EDGE_FEATS), lambda i: (i, 0)),
            pl.BlockSpec((EDGE_FEATS, HIDDEN), lambda i: (0, 0)),
            pl.BlockSpec((1, HIDDEN), lambda i: (0, 0)),
        ],
        out_specs=pl.BlockSpec((blk, HIDDEN), lambda i: (i, 0)),
        out_shape=jax.ShapeDtypeStruct((N_EDGES, HIDDEN), jnp.float32),
    )(edge_attr, We, be.reshape(1, HIDDEN))


def _dense_body(h_ref, p_ref, w1_ref, b1_ref, w2_ref, b2_ref,
                g_ref, bt_ref, o_ref):
    h = h_ref[...]
    z = h + p_ref[0:N_NODES, :] + p_ref[N_PAD:N_PAD + N_NODES, :]
    z = jnp.dot(z, w1_ref[...], preferred_element_type=jnp.float32) + b1_ref[...]
    z = jnp.maximum(z, 0.0)
    z = jnp.dot(z, w2_ref[...], preferred_element_type=jnp.float32) + b2_ref[...]
    mu = jnp.mean(z, axis=0, keepdims=True)
    var = jnp.mean(jnp.square(z - mu), axis=0, keepdims=True)
    zn = (z - mu) * lax.rsqrt(var + BN_EPS) * g_ref[...] + bt_ref[...]
    o_ref[...] = jnp.maximum(zn, 0.0) + h


def _dense_layer(h, p, W1l, b1l, W2l, b2l, gl, btl):
    spec = lambda shape: pl.BlockSpec(shape, lambda: tuple(0 for _ in shape))
    return pl.pallas_call(
        _dense_body,
        in_specs=[
            spec((N_NODES, HIDDEN)),
            spec((NC * N_PAD, HIDDEN)),
            spec((HIDDEN, HIDDEN)),
            spec((1, HIDDEN)),
            spec((HIDDEN, HIDDEN)),
            spec((1, HIDDEN)),
            spec((1, HIDDEN)),
            spec((1, HIDDEN)),
        ],
        out_specs=spec((N_NODES, HIDDEN)),
        out_shape=jax.ShapeDtypeStruct((N_NODES, HIDDEN), jnp.float32),
    )(h, p, W1l, b1l.reshape(1, HIDDEN), W2l, b2l.reshape(1, HIDDEN),
      gl.reshape(1, HIDDEN), btl.reshape(1, HIDDEN))


def kernel(x, edge_index, edge_attr, We, be, W1, b1, W2, b2, gamma, beta):
    src = edge_index[0].astype(jnp.int32)
    dst = edge_index[1].astype(jnp.int32)
    e = _edge_transform(edge_attr, We, be)

    def _layer(h, ws):
        W1l, b1l, W2l, b2l, gl, btl = ws
        p = _mp_kernel(h, e, src, dst)
        h = _dense_layer(h, p, W1l, b1l, W2l, b2l, gl, btl)
        return h, None

    h, _ = lax.scan(_layer, x, (W1, b1, W2, b2, gamma, beta))
    return h
